# baseline (speedup 1.0000x reference)
import functools

import jax
import jax.numpy as jnp
from jax.experimental import pallas as pl
from jax.experimental.pallas import tpu as pltpu

N = 2048
BLK = 512


def _attr_kernel(x_ref, w_ref, b_ref, attr_ref, nrm2_ref):
    j = pl.program_id(1)
    acc = jax.lax.dot_general(
        x_ref[...], w_ref[...], (((1,), (0,)), ((), ())),
        preferred_element_type=jnp.float32,
    )
    a = jax.nn.sigmoid(acc + b_ref[...])
    attr_ref[...] = a
    part = jnp.sum(a * a, axis=1, keepdims=True)

    @pl.when(j == 0)
    def _():
        nrm2_ref[...] = jnp.zeros_like(nrm2_ref)

    nrm2_ref[...] += part


def _gram_kernel(ai_ref, aj_ref, ni_ref, nj_ref, c_ref, maxr_ref):
    j = pl.program_id(1)
    g = jax.lax.dot_general(
        ai_ref[...], aj_ref[...], (((1,), (1,)), ((), ())),
        preferred_element_type=jnp.float32,
    )
    denom = jnp.maximum(jnp.sqrt(ni_ref[...]) * jnp.sqrt(nj_ref[...]), 1e-8)
    c = g / denom
    c_ref[...] = c
    part = jnp.max(c, axis=1, keepdims=True)

    @pl.when(j == 0)
    def _():
        maxr_ref[...] = jnp.full_like(maxr_ref, -jnp.inf)

    maxr_ref[...] = jnp.maximum(maxr_ref[...], part)


def _compute_c_maxr(x, W, b):
    nb = N // BLK
    attr, nrm2 = pl.pallas_call(
        _attr_kernel,
        grid=(nb, nb),
        in_specs=[
            pl.BlockSpec((BLK, N), lambda i, j: (i, 0)),
            pl.BlockSpec((N, BLK), lambda i, j: (0, j)),
            pl.BlockSpec((1, BLK), lambda i, j: (0, j)),
        ],
        out_specs=[
            pl.BlockSpec((BLK, BLK), lambda i, j: (i, j)),
            pl.BlockSpec((BLK, 1), lambda i, j: (i, 0)),
        ],
        out_shape=[
            jax.ShapeDtypeStruct((N, N), jnp.float32),
            jax.ShapeDtypeStruct((N, 1), jnp.float32),
        ],
    )(x, W, b.reshape(1, N))

    nrm2_row = nrm2.reshape(1, N)
    c, maxr = pl.pallas_call(
        _gram_kernel,
        grid=(nb, nb),
        in_specs=[
            pl.BlockSpec((BLK, N), lambda i, j: (i, 0)),
            pl.BlockSpec((BLK, N), lambda i, j: (j, 0)),
            pl.BlockSpec((BLK, 1), lambda i, j: (i, 0)),
            pl.BlockSpec((1, BLK), lambda i, j: (0, j)),
        ],
        out_specs=[
            pl.BlockSpec((BLK, BLK), lambda i, j: (i, j)),
            pl.BlockSpec((BLK, 1), lambda i, j: (i, 0)),
        ],
        out_shape=[
            jax.ShapeDtypeStruct((N, N), jnp.float32),
            jax.ShapeDtypeStruct((N, 1), jnp.float32),
        ],
    )(attr, attr, nrm2, nrm2_row)
    return c, maxr


def kernel(x, W, b):
    c, maxr = _compute_c_maxr(x, W, b)
    a_norm = c / maxr
    values, indices = jax.lax.top_k(a_norm, N)
    src = jnp.repeat(jnp.arange(N, dtype=jnp.int32), N)
    edge_index = jnp.stack([src, indices.reshape(-1).astype(jnp.int32)])
    return (values.reshape(-1), edge_index, a_norm)
